# baseline (speedup 1.0000x reference)
import functools

import jax
import jax.numpy as jnp
from jax import lax
from jax.experimental import pallas as pl
from jax.experimental.pallas import tpu as pltpu
from jax.experimental.pallas import tpu_sc as plsc

_GRID = 2


def _feats_body(x_ref, s_ref, cond_ref, t_ref, w_ref, wct_ref, of_ref):
    condm = jnp.mean(cond_ref[...], axis=0, keepdims=True)
    cvt = lax.dot_general(wct_ref[...], condm,
                          (((1,), (1,)), ((), ())),
                          preferred_element_type=jnp.float32)
    tt = t_ref[0, 0]
    xw = lax.dot_general(w_ref[...], x_ref[...],
                         (((0,), (0,)), ((), ())),
                         preferred_element_type=jnp.float32)
    sw = lax.dot_general(w_ref[...], s_ref[...],
                         (((0,), (0,)), ((), ())),
                         preferred_element_type=jnp.float32)
    of_ref[...] = xw * tt + cvt + jnp.tanh(sw)


def _coords_copy(n_elems):
    mesh = plsc.VectorSubcoreMesh(core_axis_name="c", subcore_axis_name="s")
    info = plsc.get_sparse_core_info()
    nw = info.num_cores * info.num_subcores
    chunk = n_elems // nw

    @functools.partial(
        pl.kernel, mesh=mesh,
        out_type=jax.ShapeDtypeStruct((n_elems,), jnp.int32),
    )
    def copy_k(src_hbm, out_hbm):
        wid = lax.axis_index("s") * info.num_cores + lax.axis_index("c")
        base = wid * chunk
        pltpu.sync_copy(src_hbm.at[pl.ds(base, chunk)],
                        out_hbm.at[pl.ds(base, chunk)])

    return copy_k


def kernel(x_t_feats, x_t_coords, tex_feats, tex_coords, shape_feats,
           shape_coords, t, cond, coords_len_list, W, Wc):
    T, D = x_t_feats.shape
    dc = x_t_coords.shape[1]
    B, DCOND = cond.shape

    xT = x_t_feats.T
    sT = shape_feats.T
    wcT = Wc.T
    t2 = t.reshape(1, 1)
    blk = T // _GRID

    ofT = pl.pallas_call(
        _feats_body,
        grid=(_GRID,),
        in_specs=[
            pl.BlockSpec((D, blk), lambda i: (0, i)),
            pl.BlockSpec((D, blk), lambda i: (0, i)),
            pl.BlockSpec((B, DCOND), lambda i: (0, 0)),
            pl.BlockSpec((1, 1), lambda i: (0, 0)),
            pl.BlockSpec((D, D), lambda i: (0, 0)),
            pl.BlockSpec((D, DCOND), lambda i: (0, 0)),
        ],
        out_specs=pl.BlockSpec((D, blk), lambda i: (0, i)),
        out_shape=jax.ShapeDtypeStruct((D, T), jnp.float32),
        compiler_params=pltpu.CompilerParams(
            dimension_semantics=("arbitrary",)),
    )(xT, sT, cond, t2, W, wcT)

    cflat = x_t_coords.T.reshape(T * dc)
    ocT = _coords_copy(T * dc)(cflat).reshape(dc, T)
    return ofT.T, ocT.T


# speedup vs baseline: 7.1126x; 7.1126x over previous
import jax
import jax.numpy as jnp
from jax import lax
from jax.experimental import pallas as pl
from jax.experimental.pallas import tpu as pltpu

_GRID = 2


def _body(x_ref, s_ref, c_ref, cond_ref, t_ref, w_ref, wct_ref,
          of_ref, oc_ref, cvt_ref):
    @pl.when(pl.program_id(0) == 0)
    def _():
        condm = jnp.mean(cond_ref[...], axis=0, keepdims=True)
        cvt_ref[...] = lax.dot_general(
            wct_ref[...], condm, (((1,), (1,)), ((), ())),
            preferred_element_type=jnp.float32)

    tt = t_ref[0, 0]
    xw = lax.dot_general(w_ref[...], x_ref[...],
                         (((0,), (0,)), ((), ())),
                         preferred_element_type=jnp.float32)
    sw = lax.dot_general(w_ref[...], s_ref[...],
                         (((0,), (0,)), ((), ())),
                         preferred_element_type=jnp.float32)
    of_ref[...] = xw * tt + cvt_ref[...] + jnp.tanh(sw)
    oc_ref[...] = c_ref[...]


def kernel(x_t_feats, x_t_coords, tex_feats, tex_coords, shape_feats,
           shape_coords, t, cond, coords_len_list, W, Wc):
    T, D = x_t_feats.shape
    dc = x_t_coords.shape[1]
    B, DCOND = cond.shape

    xT = x_t_feats.T
    sT = shape_feats.T
    cT = x_t_coords.T
    wcT = Wc.T
    t2 = t.reshape(1, 1)
    blk = T // _GRID

    ofT, ocT = pl.pallas_call(
        _body,
        grid=(_GRID,),
        in_specs=[
            pl.BlockSpec((D, blk), lambda i: (0, i)),
            pl.BlockSpec((D, blk), lambda i: (0, i)),
            pl.BlockSpec((dc, blk), lambda i: (0, i)),
            pl.BlockSpec((B, DCOND), lambda i: (0, 0)),
            pl.BlockSpec((1, 1), lambda i: (0, 0)),
            pl.BlockSpec((D, D), lambda i: (0, 0)),
            pl.BlockSpec((D, DCOND), lambda i: (0, 0)),
        ],
        out_specs=[
            pl.BlockSpec((D, blk), lambda i: (0, i)),
            pl.BlockSpec((dc, blk), lambda i: (0, i)),
        ],
        out_shape=[
            jax.ShapeDtypeStruct((D, T), jnp.float32),
            jax.ShapeDtypeStruct((dc, T), jnp.int32),
        ],
        scratch_shapes=[pltpu.VMEM((D, 1), jnp.float32)],
        compiler_params=pltpu.CompilerParams(
            dimension_semantics=("arbitrary",)),
    )(xT, sT, cT, cond, t2, W, wcT)

    return ofT.T, ocT.T


# speedup vs baseline: 7.1813x; 1.0097x over previous
import jax
import jax.numpy as jnp
from jax import lax
from jax.experimental import pallas as pl
from jax.experimental.pallas import tpu as pltpu

_GRID = 2


def _body(x_ref, s_ref, c_ref, cond_ref, t_ref, w_ref, wct_ref,
          of_ref, oc_ref):
    condm = jnp.mean(cond_ref[...], axis=0, keepdims=True)
    cvt = lax.dot_general(wct_ref[...], condm,
                          (((1,), (1,)), ((), ())),
                          preferred_element_type=jnp.float32)
    tt = t_ref[0, 0]
    xw = lax.dot_general(w_ref[...], x_ref[...],
                         (((0,), (0,)), ((), ())),
                         preferred_element_type=jnp.float32)
    sw = lax.dot_general(w_ref[...], s_ref[...],
                         (((0,), (0,)), ((), ())),
                         preferred_element_type=jnp.float32)
    of_ref[...] = xw * tt + cvt + jnp.tanh(sw)
    oc_ref[...] = c_ref[...]


def kernel(x_t_feats, x_t_coords, tex_feats, tex_coords, shape_feats,
           shape_coords, t, cond, coords_len_list, W, Wc):
    T, D = x_t_feats.shape
    dc = x_t_coords.shape[1]
    B, DCOND = cond.shape

    xT = x_t_feats.T
    sT = shape_feats.T
    cT = x_t_coords.T
    wcT = Wc.T
    t2 = t.reshape(1, 1)
    blk = T // _GRID

    ofT, ocT = pl.pallas_call(
        _body,
        grid=(_GRID,),
        in_specs=[
            pl.BlockSpec((D, blk), lambda i: (0, i)),
            pl.BlockSpec((D, blk), lambda i: (0, i)),
            pl.BlockSpec((dc, blk), lambda i: (0, i)),
            pl.BlockSpec((B, DCOND), lambda i: (0, 0)),
            pl.BlockSpec((1, 1), lambda i: (0, 0)),
            pl.BlockSpec((D, D), lambda i: (0, 0)),
            pl.BlockSpec((D, DCOND), lambda i: (0, 0)),
        ],
        out_specs=[
            pl.BlockSpec((D, blk), lambda i: (0, i)),
            pl.BlockSpec((dc, blk), lambda i: (0, i)),
        ],
        out_shape=[
            jax.ShapeDtypeStruct((D, T), jnp.float32),
            jax.ShapeDtypeStruct((dc, T), jnp.int32),
        ],
        compiler_params=pltpu.CompilerParams(
            dimension_semantics=("arbitrary",)),
    )(xT, sT, cT, cond, t2, W, wcT)

    return ofT.T, ocT.T


# speedup vs baseline: 7.2663x; 1.0118x over previous
import jax
import jax.numpy as jnp
from jax import lax
from jax.experimental import pallas as pl
from jax.experimental.pallas import tpu as pltpu

_GRID = 2


def _body(x_ref, s_ref, c_ref, cond_ref, t_ref, w_ref, wct_ref,
          of_ref, oc_ref):
    condm = jnp.mean(cond_ref[...], axis=0, keepdims=True)
    cvt = lax.dot_general(wct_ref[...], condm,
                          (((1,), (1,)), ((), ())),
                          preferred_element_type=jnp.float32)
    tt = t_ref[0, 0]
    xw = lax.dot_general(w_ref[...], x_ref[...],
                         (((0,), (0,)), ((), ())),
                         preferred_element_type=jnp.float32)
    sw = lax.dot_general(w_ref[...], s_ref[...],
                         (((0,), (0,)), ((), ())),
                         preferred_element_type=jnp.float32)
    of_ref[...] = xw * tt + cvt + jnp.tanh(sw)
    oc_ref[...] = c_ref[...]


def kernel(x_t_feats, x_t_coords, tex_feats, tex_coords, shape_feats,
           shape_coords, t, cond, coords_len_list, W, Wc):
    T, D = x_t_feats.shape
    dc = x_t_coords.shape[1]
    B, DCOND = cond.shape

    xT = x_t_feats.T
    sT = shape_feats.T
    cT = x_t_coords.T
    wcT = Wc.T
    t2 = t.reshape(1, 1)
    blk = T // _GRID

    ofT, ocT = pl.pallas_call(
        _body,
        grid=(_GRID,),
        in_specs=[
            pl.BlockSpec((D, blk), lambda i: (0, i)),
            pl.BlockSpec((D, blk), lambda i: (0, i)),
            pl.BlockSpec((dc, blk), lambda i: (0, i)),
            pl.BlockSpec((B, DCOND), lambda i: (0, 0)),
            pl.BlockSpec((1, 1), lambda i: (0, 0)),
            pl.BlockSpec((D, D), lambda i: (0, 0)),
            pl.BlockSpec((D, DCOND), lambda i: (0, 0)),
        ],
        out_specs=[
            pl.BlockSpec((D, blk), lambda i: (0, i)),
            pl.BlockSpec((dc, blk), lambda i: (0, i)),
        ],
        out_shape=[
            jax.ShapeDtypeStruct((D, T), jnp.float32),
            jax.ShapeDtypeStruct((dc, T), jnp.int32),
        ],
        compiler_params=pltpu.CompilerParams(
            dimension_semantics=("parallel",)),
    )(xT, sT, cT, cond, t2, W, wcT)

    return ofT.T, ocT.T
